# Initial kernel scaffold; baseline (speedup 1.0000x reference)
import jax
import jax.numpy as jnp
from jax.experimental import pallas as pl


def kernel(images, gt_boxes, gt_classes, anchor_boxes):
    raise NotImplementedError("write your pallas kernel here")


# speedup vs baseline: 28.2585x; 28.2585x over previous
import jax
import jax.numpy as jnp
from jax.experimental import pallas as pl

_NEG_T = 0.4
_POS_T = 0.5
_N_PAD = 128


def _encode_kernel(anch_ref, gtr_ref, gtc_ref, box_ref, cls_ref, *, inv_h, inv_w):
    a0 = anch_ref[0:1, :]
    a1 = anch_ref[1:2, :]
    a2 = anch_ref[2:3, :]
    a3 = anch_ref[3:4, :]
    A2x = a0 + a2
    A2y = a1 + a3
    area_a = a2 * a3

    gt_cols = gtc_ref[0]
    X1 = gt_cols[:, 0:1]
    Y1 = gt_cols[:, 1:2]
    GW = gt_cols[:, 2:3]
    GH = gt_cols[:, 3:4]
    X2 = X1 + GW
    Y2 = Y1 + GH
    area_g = GW * GH

    ix = jnp.maximum(jnp.minimum(A2x, X2) - jnp.maximum(a0, X1), 0.0)
    iy = jnp.maximum(jnp.minimum(A2y, Y2) - jnp.maximum(a1, Y1), 0.0)
    inter = ix * iy
    union = area_a + area_g - inter
    iou = jnp.where(union > 0.0, inter / jnp.where(union > 0.0, union, 1.0), 0.0)

    mx = jnp.max(iou, axis=0, keepdims=True)
    iota = jax.lax.broadcasted_iota(jnp.int32, iou.shape, 0)
    cand = jnp.where(iou == mx, iota, _N_PAD)
    fidx = jnp.min(cand, axis=0, keepdims=True)
    onehot = (iota == fidx).astype(jnp.float32)

    gt_rows = gtr_ref[0]
    gx_r = gt_rows[0:1]
    gy_r = gt_rows[1:2]
    gw_r = gt_rows[2:3]
    gh_r = gt_rows[3:4]
    gc_r = gt_rows[4:5]
    tbl = jnp.concatenate([gx_r, gy_r, gx_r + gw_r, gy_r + gh_r, gc_r], axis=0)
    m5 = jax.lax.dot_general(tbl, onehot, (((1,), (0,)), ((), ())),
                             preferred_element_type=jnp.float32)
    gx = m5[0:1]
    gy = m5[1:2]
    gxw = m5[2:3]
    gyh = m5[3:4]
    gc = m5[4:5]

    cx0 = (a0 + a2) * 0.5
    cy0 = (a1 + a3) * 0.5
    r0 = 1.0 / (a2 - a0)
    r1 = 1.0 / (a3 - a1)
    p10 = (cx0 - gy * inv_h) * r0
    p11 = (cy0 - gx * inv_w) * r1
    p20 = (gyh * inv_h - cx0) * r0
    p21 = (gxw * inv_w - cy0) * r1

    cls = jnp.where(mx >= _POS_T, gc,
                    jnp.where(mx >= _NEG_T, -2.0, -1.0))

    nan = (jnp.isnan(p10) | jnp.isnan(p11) | jnp.isnan(p20)
           | jnp.isnan(p21) | jnp.isnan(cls))
    p10 = jnp.where(nan, -2.0, p10)
    p11 = jnp.where(nan, -2.0, p11)
    p20 = jnp.where(nan, -2.0, p20)
    p21 = jnp.where(nan, -2.0, p21)
    cls = jnp.where(nan, -2.0, cls)

    box_ref[0] = jnp.concatenate([p10, p11, p20, p21], axis=0)
    cls_ref[0] = cls


def kernel(images, gt_boxes, gt_classes, anchor_boxes):
    B, N = gt_boxes.shape[0], gt_boxes.shape[1]
    M = anchor_boxes.shape[0]
    H, W = images.shape[1], images.shape[2]

    anch_t = anchor_boxes.T
    gt5 = jnp.concatenate([gt_boxes, gt_classes], axis=-1)
    gt_cols = jnp.pad(gt5, ((0, 0), (0, _N_PAD - N), (0, 3)))
    gt_rows = jnp.transpose(gt_cols, (0, 2, 1))

    import functools
    body = functools.partial(_encode_kernel, inv_h=1.0 / H, inv_w=1.0 / W)
    box_t, cls_t = pl.pallas_call(
        body,
        grid=(B,),
        in_specs=[
            pl.BlockSpec((4, M), lambda b: (0, 0)),
            pl.BlockSpec((1, 8, _N_PAD), lambda b: (b, 0, 0)),
            pl.BlockSpec((1, _N_PAD, 8), lambda b: (b, 0, 0)),
        ],
        out_specs=[
            pl.BlockSpec((1, 4, M), lambda b: (b, 0, 0)),
            pl.BlockSpec((1, 1, M), lambda b: (b, 0, 0)),
        ],
        out_shape=[
            jax.ShapeDtypeStruct((B, 4, M), jnp.float32),
            jax.ShapeDtypeStruct((B, 1, M), jnp.float32),
        ],
    )(anch_t, gt_rows, gt_cols)
    return jnp.transpose(box_t, (0, 2, 1)), cls_t[:, 0, :]
